# baseline (speedup 1.0000x reference)
import functools

import jax
import jax.numpy as jnp
from jax import lax
from jax.experimental import pallas as pl
from jax.experimental.pallas import tpu as pltpu
from jax.experimental.pallas import tpu_sc as plsc

NC = 2
NS = 16
NW = NC * NS
L = 16


@functools.lru_cache(maxsize=None)
def _build_relayout(vocab, dim):
    full_tiles = vocab // 128
    per_w = full_tiles // NW
    extra = full_tiles - per_w * NW
    rem_cols = vocab - full_tiles * 128
    rows_out = vocab * dim // 128
    mesh = plsc.VectorSubcoreMesh(core_axis_name="c", subcore_axis_name="s")

    rem_rows = rem_cols * dim // 128

    @functools.partial(
        pl.kernel,
        mesh=mesh,
        out_type=jax.ShapeDtypeStruct((rows_out, 128), jnp.float32),
        scratch_types=[
            pltpu.VMEM((2, dim, 128), jnp.float32),
            pltpu.VMEM((2, dim, 128), jnp.float32),
            pltpu.SemaphoreType.DMA,
            pltpu.SemaphoreType.DMA,
        ],
        compiler_params=pltpu.CompilerParams(use_tc_tiling_on_sc=True,
                                             needs_layout_passes=False),
    )
    def relayout_kernel(tt_hbm, tail_hbm, out_hbm, slab, tpose, ssem, wsem):
        wid = lax.axis_index("s") * NC + lax.axis_index("c")
        base = wid * per_w
        lanes = lax.iota(jnp.int32, L)

        def fire_slabs(p, c, width):
            for r in range(dim // 8):
                pltpu.async_copy(
                    tt_hbm.at[pl.ds(8 * r, 8), pl.ds(c * 128, width)],
                    slab.at[p, pl.ds(8 * r, 8), pl.ds(0, width)], ssem)

        def wait_slabs(p, c, width):
            for r in range(dim // 8):
                pltpu.make_async_copy(
                    tt_hbm.at[pl.ds(8 * r, 8), pl.ds(c * 128, width)],
                    slab.at[p, pl.ds(8 * r, 8), pl.ds(0, width)], ssem).wait()

        def transpose(p):
            def tm(mq, _):
                for mi in range(4):
                    rot = jnp.bitwise_and(lanes + (mq * 4 + mi), L - 1)
                    for d0 in range(0, dim, L):
                        dvec = rot + d0
                        for j0 in range(0, 128, L):
                            jvec = lanes + j0
                            v = plsc.load_gather(slab.at[p], [dvec, jvec])
                            flat = jvec * dim + dvec
                            plsc.store_scatter(
                                tpose.at[p],
                                [lax.shift_right_logical(flat, 7),
                                 jnp.bitwise_and(flat, 127)], v)
                return 0

            lax.fori_loop(0, L // 4, tm, 0)

        def fire_write(p, c):
            pltpu.async_copy(tpose.at[p], out_hbm.at[pl.ds(c * dim, dim)],
                             wsem)

        def wait_write(p, c):
            pltpu.make_async_copy(tpose.at[p], out_hbm.at[pl.ds(c * dim, dim)],
                                  wsem).wait()

        fire_slabs(0, base, 128)
        fire_slabs(1, base + 1, 128)

        def pair(i, _):
            for p in (0, 1):
                u = 2 * i + p
                c = base + u
                wait_slabs(p, c, 128)

                @pl.when(u >= 2)
                def _():
                    wait_write(p, c - 2)
                transpose(p)

                @pl.when(u + 2 < per_w)
                def _():
                    fire_slabs(p, c + 2, 128)
                fire_write(p, c)
            return 0

        lax.fori_loop(0, per_w // 2, pair, 0)
        wait_write(0, base + per_w - 2)
        wait_write(1, base + per_w - 1)

        if extra:
            @pl.when(wid < extra)
            def _():
                c = per_w * NW + wid
                for r in range(dim // 8):
                    pltpu.sync_copy(
                        tt_hbm.at[pl.ds(8 * r, 8), pl.ds(c * 128, 128)],
                        slab.at[0, pl.ds(8 * r, 8)])
                transpose(0)
                pltpu.sync_copy(tpose.at[0], out_hbm.at[pl.ds(c * dim, dim)])

        if rem_cols:
            @pl.when(wid == extra)
            def _():
                pltpu.sync_copy(tail_hbm, tpose.at[0, pl.ds(0, rem_rows)])
                pltpu.sync_copy(
                    tpose.at[0, pl.ds(0, rem_rows)],
                    out_hbm.at[pl.ds(full_tiles * dim, rem_rows)])

    return relayout_kernel


@functools.lru_cache(maxsize=None)
def _build_gather(batch, fields, vocab, dim):
    bl = 128
    nbt = batch // bl
    bt_per_w = nbt // NW
    dt = dim // 8
    blk_idx = bl * fields
    mesh = plsc.VectorSubcoreMesh(core_axis_name="c", subcore_axis_name="s")

    @functools.partial(
        pl.kernel,
        mesh=mesh,
        out_type=jax.ShapeDtypeStruct((fields, dt, nbt, 8, bl), jnp.float32),
        scratch_types=[
            pltpu.VMEM((blk_idx,), jnp.int32),
            pltpu.VMEM((4, bl), jnp.int32),
            pltpu.VMEM((4, bl, dim), jnp.float32),
            pltpu.VMEM((4, dt, 8, bl), jnp.float32),
            pltpu.SemaphoreType.DMA,
            pltpu.SemaphoreType.DMA,
        ],
        compiler_params=pltpu.CompilerParams(use_tc_tiling_on_sc=False,
                                             needs_layout_passes=False),
    )
    def gather_kernel(table_hbm, idx_hbm, out_hbm,
                      idxb, idxf, rows, tbuf, gsem, wsem):
        wid = lax.axis_index("s") * NC + lax.axis_index("c")
        lanes = lax.iota(jnp.int32, L)

        def extract_idx(p, f):
            for j0 in range(bl // L):
                pos = (lanes + (j0 * L)) * fields + f
                v = plsc.load_gather(idxb, [pos])
                idxf[p, pl.ds(j0 * L, L)] = v

        def fire_gather(p):
            return pltpu.async_copy(table_hbm.at[idxf.at[p]], rows.at[p], gsem)

        def wait_gather(p):
            pltpu.make_async_copy(table_hbm.at[idxf.at[p]], rows.at[p],
                                  gsem).wait()

        def transpose(p):
            def tm(mq, _):
                for mi in range(4):
                    rot = jnp.bitwise_and(lanes + (mq * 4 + mi), L - 1)
                    for d0 in range(0, dim, L):
                        dvec = rot + d0
                        tvec = lax.shift_right_logical(dvec, 3)
                        svec = jnp.bitwise_and(dvec, 7)
                        for j0 in range(0, bl, L):
                            jvec = lanes + j0
                            v = plsc.load_gather(rows.at[p], [jvec, dvec])
                            plsc.store_scatter(tbuf.at[p], [tvec, svec, jvec],
                                               v)
                return 0

            lax.fori_loop(0, L // 4, tm, 0)

        def fire_write(p, f, bt):
            for t in range(dt):
                pltpu.async_copy(tbuf.at[p, t], out_hbm.at[f, t, bt], wsem)

        def wait_write(p, f, bt):
            for t in range(dt):
                pltpu.make_async_copy(tbuf.at[p, t], out_hbm.at[f, t, bt],
                                      wsem).wait()

        def per_block(u, _):
            bt = wid * bt_per_w + u
            pltpu.sync_copy(idx_hbm.at[pl.ds(bt * blk_idx, blk_idx)], idxb)
            for f in range(3):
                extract_idx(f, f)
                fire_gather(f)

            def step(p, f):
                fire_p = (p + 3) % 4
                static = isinstance(f, int)
                wait_gather(p)
                if static:
                    if f >= 4:
                        wait_write(p, f - 4, bt)
                else:
                    @pl.when(f >= 4)
                    def _():
                        wait_write(p, f - 4, bt)
                transpose(p)
                if static:
                    if f + 3 < fields:
                        extract_idx(fire_p, f + 3)
                        fire_gather(fire_p)
                else:
                    @pl.when(f + 3 < fields)
                    def _():
                        extract_idx(fire_p, f + 3)
                        fire_gather(fire_p)
                fire_write(p, f, bt)

            def quad(i, _):
                f0 = 4 * i
                for p in range(4):
                    step(p, f0 + p)
                return 0

            lax.fori_loop(0, fields // 4, quad, 0)
            for f in range((fields // 4) * 4, fields):
                step(f % 4, f)
            for f in range(fields - 4, fields):
                wait_write(f % 4, f, bt)
            return 0

        lax.fori_loop(0, bt_per_w, per_block, 0)

    return gather_kernel


def kernel(indices, table):
    batch, fields = indices.shape
    vocab, dim = table.shape
    idx_flat = indices.reshape(batch * fields).astype(jnp.int32)
    full_vocab = (vocab // 128) * 128
    tail = table[full_vocab:].reshape((vocab - full_vocab) * dim // 128, 128)
    t128 = _build_relayout(vocab, dim)(table.T, tail)
    tbl_lin = t128.reshape(vocab, dim)
    gather = _build_gather(batch, fields, vocab, dim)
    p5 = gather(tbl_lin, idx_flat)
    out_t = p5.transpose(0, 1, 3, 2, 4).reshape(fields, dim, batch)
    return out_t.transpose(2, 0, 1)


# speedup vs baseline: 1.1579x; 1.1579x over previous
import functools

import jax
import jax.numpy as jnp
from jax import lax
from jax.experimental import pallas as pl
from jax.experimental.pallas import tpu as pltpu
from jax.experimental.pallas import tpu_sc as plsc

NC = 2
NS = 16
NW = NC * NS
L = 16


@functools.lru_cache(maxsize=None)
def _build_relayout(vocab, dim):
    full_tiles = vocab // 128
    per_w = full_tiles // NW
    extra = full_tiles - per_w * NW
    rem_cols = vocab - full_tiles * 128
    rows_out = vocab * dim // 128
    mesh = plsc.VectorSubcoreMesh(core_axis_name="c", subcore_axis_name="s")

    rem_rows = rem_cols * dim // 128

    @functools.partial(
        pl.kernel,
        mesh=mesh,
        out_type=jax.ShapeDtypeStruct((rows_out, 128), jnp.float32),
        scratch_types=[
            pltpu.VMEM((2, dim, 128), jnp.float32),
            pltpu.VMEM((2, dim, 128), jnp.float32),
            pltpu.SemaphoreType.DMA,
            pltpu.SemaphoreType.DMA,
        ],
        compiler_params=pltpu.CompilerParams(use_tc_tiling_on_sc=True,
                                             needs_layout_passes=False),
    )
    def relayout_kernel(tt_hbm, tail_hbm, out_hbm, slab, tpose, ssem, wsem):
        wid = lax.axis_index("s") * NC + lax.axis_index("c")
        base = wid * per_w
        lanes = lax.iota(jnp.int32, L)

        def fire_slabs(p, c, width):
            for r in range(dim // 8):
                pltpu.async_copy(
                    tt_hbm.at[pl.ds(8 * r, 8), pl.ds(c * 128, width)],
                    slab.at[p, pl.ds(8 * r, 8), pl.ds(0, width)], ssem)

        def wait_slabs(p, c, width):
            for r in range(dim // 8):
                pltpu.make_async_copy(
                    tt_hbm.at[pl.ds(8 * r, 8), pl.ds(c * 128, width)],
                    slab.at[p, pl.ds(8 * r, 8), pl.ds(0, width)], ssem).wait()

        def transpose(p):
            def tm(mq, _):
                for mi in range(4):
                    rot = jnp.bitwise_and(lanes + (mq * 4 + mi), L - 1)
                    for d0 in range(0, dim, L):
                        dvec = rot + d0
                        for j0 in range(0, 128, L):
                            jvec = lanes + j0
                            v = plsc.load_gather(slab.at[p], [dvec, jvec])
                            flat = jvec * dim + dvec
                            plsc.store_scatter(
                                tpose.at[p],
                                [lax.shift_right_logical(flat, 7),
                                 jnp.bitwise_and(flat, 127)], v)
                return 0

            lax.fori_loop(0, L // 4, tm, 0)

        def fire_write(p, c):
            pltpu.async_copy(tpose.at[p], out_hbm.at[pl.ds(c * dim, dim)],
                             wsem)

        def wait_write(p, c):
            pltpu.make_async_copy(tpose.at[p], out_hbm.at[pl.ds(c * dim, dim)],
                                  wsem).wait()

        fire_slabs(0, base, 128)
        fire_slabs(1, base + 1, 128)

        def pair(i, _):
            for p in (0, 1):
                u = 2 * i + p
                c = base + u
                wait_slabs(p, c, 128)

                @pl.when(u >= 2)
                def _():
                    wait_write(p, c - 2)
                transpose(p)

                @pl.when(u + 2 < per_w)
                def _():
                    fire_slabs(p, c + 2, 128)
                fire_write(p, c)
            return 0

        lax.fori_loop(0, per_w // 2, pair, 0)
        wait_write(0, base + per_w - 2)
        wait_write(1, base + per_w - 1)

        if extra:
            @pl.when(wid < extra)
            def _():
                c = per_w * NW + wid
                for r in range(dim // 8):
                    pltpu.sync_copy(
                        tt_hbm.at[pl.ds(8 * r, 8), pl.ds(c * 128, 128)],
                        slab.at[0, pl.ds(8 * r, 8)])
                transpose(0)
                pltpu.sync_copy(tpose.at[0], out_hbm.at[pl.ds(c * dim, dim)])

        if rem_cols:
            @pl.when(wid == extra)
            def _():
                pltpu.sync_copy(tail_hbm, tpose.at[0, pl.ds(0, rem_rows)])
                pltpu.sync_copy(
                    tpose.at[0, pl.ds(0, rem_rows)],
                    out_hbm.at[pl.ds(full_tiles * dim, rem_rows)])

    return relayout_kernel


@functools.lru_cache(maxsize=None)
def _build_gather(batch, fields, vocab, dim):
    bl = 128
    nbt = batch // bl
    bt_per_w = nbt // NW
    dt = dim // 8
    blk_idx = bl * fields
    mesh = plsc.VectorSubcoreMesh(core_axis_name="c", subcore_axis_name="s")

    @functools.partial(
        pl.kernel,
        mesh=mesh,
        out_type=jax.ShapeDtypeStruct((fields, dt, nbt, 8, bl), jnp.float32),
        scratch_types=[
            pltpu.VMEM((blk_idx,), jnp.int32),
            pltpu.VMEM((2, bl), jnp.int32),
            pltpu.VMEM((2, bl, dim), jnp.float32),
            pltpu.VMEM((2, dt, 8, bl), jnp.float32),
            pltpu.SemaphoreType.DMA,
            pltpu.SemaphoreType.DMA,
        ],
        compiler_params=pltpu.CompilerParams(use_tc_tiling_on_sc=False,
                                             needs_layout_passes=False),
    )
    def gather_kernel(table_hbm, idx_hbm, out_hbm,
                      idxb, idxf, rows, tbuf, gsem, wsem):
        wid = lax.axis_index("s") * NC + lax.axis_index("c")
        lanes = lax.iota(jnp.int32, L)

        def extract_idx(p, f):
            for j0 in range(bl // L):
                pos = (lanes + (j0 * L)) * fields + f
                v = plsc.load_gather(idxb, [pos])
                idxf[p, pl.ds(j0 * L, L)] = v

        def fire_gather(p):
            return pltpu.async_copy(table_hbm.at[idxf.at[p]], rows.at[p], gsem)

        def wait_gather(p):
            pltpu.make_async_copy(table_hbm.at[idxf.at[p]], rows.at[p],
                                  gsem).wait()

        def transpose(p):
            def tm(mq, _):
                for mi in range(4):
                    rot = jnp.bitwise_and(lanes + (mq * 4 + mi), L - 1)
                    for d0 in range(0, dim, L):
                        dvec = rot + d0
                        tvec = lax.shift_right_logical(dvec, 3)
                        svec = jnp.bitwise_and(dvec, 7)
                        for j0 in range(0, bl, L):
                            jvec = lanes + j0
                            v = plsc.load_gather(rows.at[p], [jvec, dvec])
                            plsc.store_scatter(tbuf.at[p], [tvec, svec, jvec],
                                               v)
                return 0

            lax.fori_loop(0, L // 4, tm, 0)

        def fire_write(p, f, bt):
            for t in range(dt):
                pltpu.async_copy(tbuf.at[p, t], out_hbm.at[f, t, bt], wsem)

        def wait_write(p, f, bt):
            for t in range(dt):
                pltpu.make_async_copy(tbuf.at[p, t], out_hbm.at[f, t, bt],
                                      wsem).wait()

        def per_block(u, _):
            bt = wid * bt_per_w + u
            pltpu.sync_copy(idx_hbm.at[pl.ds(bt * blk_idx, blk_idx)], idxb)
            extract_idx(0, 0)
            fire_gather(0)
            extract_idx(1, 1)
            fire_gather(1)

            def pair(i, _):
                f0 = 2 * i
                for p, f in ((0, f0), (1, f0 + 1)):
                    wait_gather(p)

                    @pl.when(f >= 2)
                    def _():
                        wait_write(p, f - 2, bt)
                    transpose(p)

                    @pl.when(f + 2 < fields)
                    def _():
                        extract_idx(p, f + 2)
                        fire_gather(p)
                    fire_write(p, f, bt)
                return 0

            lax.fori_loop(0, fields // 2, pair, 0)
            wait_write(0, fields - 2, bt)
            wait_write(1, fields - 1, bt)
            return 0

        lax.fori_loop(0, bt_per_w, per_block, 0)

    return gather_kernel


def kernel(indices, table):
    batch, fields = indices.shape
    vocab, dim = table.shape
    idx_flat = indices.reshape(batch * fields).astype(jnp.int32)
    full_vocab = (vocab // 128) * 128
    tail = table[full_vocab:].reshape((vocab - full_vocab) * dim // 128, 128)
    t128 = _build_relayout(vocab, dim)(table.T, tail)
    tbl_lin = t128.reshape(vocab, dim)
    gather = _build_gather(batch, fields, vocab, dim)
    p5 = gather(tbl_lin, idx_flat)
    out_t = p5.transpose(0, 1, 3, 2, 4).reshape(fields, dim, batch)
    return out_t.transpose(2, 0, 1)
